# baseline (speedup 1.0000x reference)
import functools

import jax
import jax.numpy as jnp
from jax import lax
from jax.experimental import pallas as pl
from jax.experimental.pallas import tpu as pltpu
from jax.experimental.pallas import tpu_sc as plsc

N = 10000
NP = 10240
DF = 128
HG = 128
HN = 512
NCLS = 16
T = 4
E = 320000

NCORE = 2
NSUB = 16
NW = NCORE * NSUB
EPW = E // NW
CH = 2000
NCHUNK = EPW // CH
CHP = 160
NCHUNKP = E // CHP
ROWS_PER_SUB = NP // NSUB


def _zero_vmem_1d(ref, nwords):
    zero = jnp.zeros((16,), jnp.float32)

    def body(i, _):
        ref[pl.ds(i * 16, 16)] = zero
        return 0

    lax.fori_loop(0, nwords // 16, body, 0)


def _sc_degree(dst0, dst1, dst2, dst3, out_hbm, ones_v, idx_a, idx_b, zbuf,
               a0, a1, a2, a3, sem_a, sem_b):
    cid = lax.axis_index("c")
    sid = lax.axis_index("s")
    wid = sid * NCORE + cid
    accs = [a0, a1, a2, a3]
    dsts = [dst0, dst1, dst2, dst3]
    idxs = (idx_a, idx_b)
    sems = (sem_a, sem_b)

    one = jnp.full((16,), 1.0, jnp.float32)

    def fill_ones(i, _):
        ones_v[pl.ds(i * 16, 16)] = one
        return 0

    lax.fori_loop(0, CH // 16, fill_ones, 0)
    _zero_vmem_1d(zbuf, ROWS_PER_SUB)

    for g in range(T):
        pltpu.sync_copy(zbuf, accs[g].at[pl.ds(sid * ROWS_PER_SUB, ROWS_PER_SUB)])
    plsc.subcore_barrier()

    for g in range(T):
        dst_hbm = dsts[g]
        acc_g = accs[g]

        def load(j, b, dst_hbm=dst_hbm):
            @pl.when(j < NCHUNK)
            def _():
                off = wid * EPW + j * CH
                pltpu.sync_copy(dst_hbm.at[pl.ds(off, CH)], idxs[b])

        def sstart(b, acc_g=acc_g):
            pltpu.async_copy(ones_v, acc_g.at[idxs[b]], sems[b], add=True)

        def swait(b, acc_g=acc_g):
            pltpu.make_async_copy(ones_v, acc_g.at[idxs[b]], sems[b]).wait()

        load(0, 0)

        def outer(j2, _):
            for b in range(2):
                j = j2 * 2 + b

                @pl.when(j < NCHUNK)
                def _():
                    sstart(b)

                @pl.when(jnp.logical_and(j >= 1, j <= NCHUNK))
                def _():
                    swait(1 - b)

                load(j + 1, 1 - b)
            return 0

        lax.fori_loop(0, NCHUNK // 2 + 1, outer, 0)

    plsc.subcore_barrier()
    for g in range(T):
        sl = pl.ds(sid * ROWS_PER_SUB, ROWS_PER_SUB)
        pltpu.sync_copy(accs[g].at[sl], out_hbm.at[cid, g, sl])


_degree_call = functools.partial(
    pl.kernel,
    out_type=jax.ShapeDtypeStruct((NCORE, T, NP), jnp.float32),
    mesh=plsc.VectorSubcoreMesh(core_axis_name="c", subcore_axis_name="s"),
    scratch_types=[
        pltpu.VMEM((CH,), jnp.float32),
        pltpu.VMEM((CH,), jnp.int32),
        pltpu.VMEM((CH,), jnp.int32),
        pltpu.VMEM((ROWS_PER_SUB,), jnp.float32),
    ] + [pltpu.VMEM_SHARED((NP,), jnp.float32) for _ in range(T)]
      + [pltpu.SemaphoreType.DMA, pltpu.SemaphoreType.DMA],
)(_sc_degree)


HH = HG


NIDX = 3


def _sc_prop(hp_hbm, src_hbm, dst_hbm, out_hbm,
             src0, src1, src2, dst0, dst1, dst2, rows0, rows1, zrow, acc,
             g0, g1, is0, is1, is2, id0, id1, id2):
    cid = lax.axis_index("c")
    sid = lax.axis_index("s")
    wid = sid * NCORE + cid

    zero = jnp.zeros((16,), jnp.float32)

    def zbody(i, _):
        zrow[i // (HG // 16), pl.ds((i % (HG // 16)) * 16, 16)] = zero
        return 0

    lax.fori_loop(0, 16 * (HG // 16), zbody, 0)
    base = sid * ROWS_PER_SUB
    for i in range(ROWS_PER_SUB // 16):
        pltpu.sync_copy(zrow, acc.at[pl.ds(base + i * 16, 16), :])
    plsc.subcore_barrier()

    nw = NCHUNKP // NW + jnp.where(wid < NCHUNKP % NW, 1, 0)
    srcs = (src0, src1, src2)
    dsts = (dst0, dst1, dst2)
    rows = (rows0, rows1)
    gsem = (g0, g1)
    isem = (is0, is1, is2)
    dsem = (id0, id1, id2)

    def idx_start(i, k):
        @pl.when(i < nw)
        def _():
            off = (wid + i * NW) * CHP
            pltpu.async_copy(src_hbm.at[pl.ds(off, CHP)], srcs[k], isem[k])
            pltpu.async_copy(dst_hbm.at[pl.ds(off, CHP)], dsts[k], dsem[k])

    def idx_wait_src(i, k):
        off = (wid + i * NW) * CHP
        pltpu.make_async_copy(src_hbm.at[pl.ds(off, CHP)], srcs[k],
                              isem[k]).wait()

    def idx_wait_dst(i, k):
        off = (wid + i * NW) * CHP
        pltpu.make_async_copy(dst_hbm.at[pl.ds(off, CHP)], dsts[k],
                              dsem[k]).wait()

    def gather_start(i, b, k):
        @pl.when(i < nw)
        def _():
            idx_wait_src(i, k)
            pltpu.async_copy(hp_hbm.at[srcs[k]], rows[b], gsem[b])

    for k in range(NIDX):
        idx_start(k, k)
    gather_start(0, 0, 0)

    def slot(i, b, k):
        @pl.when(i < nw)
        def _():
            pltpu.make_async_copy(hp_hbm.at[srcs[k]], rows[b], gsem[b]).wait()
            idx_wait_dst(i, k)
            pltpu.sync_copy(rows[b], acc.at[dsts[k]], add=True)

        idx_start(i + NIDX, k)
        gather_start(i + 1, 1 - b, (k + 1) % NIDX)

    def outer(i0, _):
        for off in range(6):
            i = i0 * 6 + off
            slot(i, off % 2, off % 3)
        return 0

    nouter = (NCHUNKP // NW + 1 + 5) // 6
    lax.fori_loop(0, nouter, outer, 0)

    plsc.subcore_barrier()
    sl = pl.ds(base, ROWS_PER_SUB)
    pltpu.sync_copy(acc.at[sl, :], out_hbm.at[cid, sl, :])


_prop_call = functools.partial(
    pl.kernel,
    out_type=jax.ShapeDtypeStruct((NCORE, NP, HG), jnp.float32),
    mesh=plsc.VectorSubcoreMesh(core_axis_name="c", subcore_axis_name="s"),
    scratch_types=[
        pltpu.VMEM((CHP,), jnp.int32),
        pltpu.VMEM((CHP,), jnp.int32),
        pltpu.VMEM((CHP,), jnp.int32),
        pltpu.VMEM((CHP,), jnp.int32),
        pltpu.VMEM((CHP,), jnp.int32),
        pltpu.VMEM((CHP,), jnp.int32),
        pltpu.VMEM((CHP, HG), jnp.float32),
        pltpu.VMEM((CHP, HG), jnp.float32),
        pltpu.VMEM((16, HG), jnp.float32),
        pltpu.VMEM_SHARED((NP, HG), jnp.float32),
    ] + [pltpu.SemaphoreType.DMA for _ in range(8)],
)(_sc_prop)


BM = 1024


def _dinv_body(degp_ref, d0, d1, d2, d3):
    deg = degp_ref[0] + degp_ref[1] + 1.0
    dinv = lax.rsqrt(deg)
    outs = [d0, d1, d2, d3]
    for g in range(T):
        outs[g][...] = dinv[g].reshape(NP, 1)


def _dinv_call(deg_partials):
    return pl.pallas_call(
        _dinv_body,
        out_shape=[jax.ShapeDtypeStruct((NP, 1), jnp.float32)] * T,
    )(deg_partials)


def _mm1_body(x_ref, w_ref, dinv_ref, out_ref):
    y = jnp.dot(x_ref[...], w_ref[...], preferred_element_type=jnp.float32)
    out_ref[...] = y * dinv_ref[...]


def _mm1_call(x, w, dinv):
    grid = (NP // BM,)
    return pl.pallas_call(
        _mm1_body,
        grid=grid,
        in_specs=[
            pl.BlockSpec((BM, DF), lambda i: (i, 0)),
            pl.BlockSpec((DF, HG), lambda i: (0, 0)),
            pl.BlockSpec((BM, 1), lambda i: (i, 0)),
        ],
        out_specs=pl.BlockSpec((BM, HG), lambda i: (i, 0)),
        out_shape=jax.ShapeDtypeStruct((NP, HG), jnp.float32),
    )(x, w, dinv)


def _layer2_body(part_ref, hp_ref, dinv_ref, w_ref, out_ref):
    dinv = dinv_ref[...]
    x = (part_ref[0] + part_ref[1] + hp_ref[...]) * dinv
    x = jnp.maximum(x, 0.0)
    y = jnp.dot(x, w_ref[...], preferred_element_type=jnp.float32)
    out_ref[...] = y * dinv


def _layer2_call(partials, hp, dinv, w):
    grid = (NP // BM,)
    return pl.pallas_call(
        _layer2_body,
        grid=grid,
        in_specs=[
            pl.BlockSpec((NCORE, BM, HH), lambda i: (0, i, 0)),
            pl.BlockSpec((BM, HG), lambda i: (i, 0)),
            pl.BlockSpec((BM, 1), lambda i: (i, 0)),
            pl.BlockSpec((HG, HG), lambda i: (0, 0)),
        ],
        out_specs=pl.BlockSpec((BM, HG), lambda i: (i, 0)),
        out_shape=jax.ShapeDtypeStruct((NP, HG), jnp.float32),
    )(partials, hp, dinv, w)


def _gates_body(part_ref, hp_ref, dinv_ref, wih_ref, bih_ref, out_ref):
    x = (part_ref[0] + part_ref[1] + hp_ref[...]) * dinv_ref[...]
    out_ref[...] = (
        jnp.dot(x, wih_ref[...], preferred_element_type=jnp.float32)
        + bih_ref[...]
    )


def _gates_call(partials, hp, dinv, wih_t, bih):
    grid = (NP // BM,)
    return pl.pallas_call(
        _gates_body,
        grid=grid,
        in_specs=[
            pl.BlockSpec((NCORE, BM, HH), lambda i: (0, i, 0)),
            pl.BlockSpec((BM, HG), lambda i: (i, 0)),
            pl.BlockSpec((BM, 1), lambda i: (i, 0)),
            pl.BlockSpec((HG, 4 * HG), lambda i: (0, 0)),
            pl.BlockSpec((1, 4 * HG), lambda i: (0, 0)),
        ],
        out_specs=pl.BlockSpec((BM, 4 * HG), lambda i: (i, 0)),
        out_shape=jax.ShapeDtypeStruct((NP, 4 * HG), jnp.float32),
    )(partials, hp, dinv, wih_t, bih)


def _log_softmax(x):
    m = jnp.max(x, axis=-1, keepdims=True)
    s = x - m
    return s - jnp.log(jnp.sum(jnp.exp(s), axis=-1, keepdims=True))


SBM = 400


def _static_body(sx_ref, w1_ref, b1_ref, w2_ref, b2_ref, out_ref):
    acc = jnp.dot(sx_ref[...], w1_ref[...], preferred_element_type=jnp.float32)
    h = jnp.maximum(acc + b1_ref[...], 0.0)
    y = jnp.dot(h, w2_ref[...], preferred_element_type=jnp.float32)
    y = y + b2_ref[...]
    out_ref[...] = _log_softmax(y)


def _static_call(s_x, w1, b1, w2, b2):
    grid = (N // SBM,)
    return pl.pallas_call(
        _static_body,
        grid=grid,
        in_specs=[
            pl.BlockSpec((SBM, N), lambda i: (i, 0)),
            pl.BlockSpec((N, HN), lambda i: (0, 0)),
            pl.BlockSpec((1, HN), lambda i: (0, 0)),
            pl.BlockSpec((HN, HG), lambda i: (0, 0)),
            pl.BlockSpec((1, HG), lambda i: (0, 0)),
        ],
        out_specs=pl.BlockSpec((SBM, HG), lambda i: (i, 0)),
        out_shape=jax.ShapeDtypeStruct((N, HG), jnp.float32),
    )(s_x, w1, b1, w2, b2)


def _sigmoid(x):
    return 1.0 / (1.0 + jnp.exp(-x))


def _lstm_body(g0, g1, g2, g3, whh_ref, bhh_ref, h0_ref, c0_ref, xs_ref,
               fw_ref, fb_ref, out_ref):
    hh = h0_ref[...]
    cc = c0_ref[...]
    whh = whh_ref[...]
    bhh = bhh_ref[...]
    for g_ref in (g0, g1, g2, g3):
        gates = g_ref[...] + jnp.dot(hh, whh,
                                     preferred_element_type=jnp.float32) + bhh
        i_g = gates[:, 0:HG]
        f_g = gates[:, HG:2 * HG]
        g_g = gates[:, 2 * HG:3 * HG]
        o_g = gates[:, 3 * HG:4 * HG]
        cc = _sigmoid(f_g) * cc + _sigmoid(i_g) * jnp.tanh(g_g)
        hh = _sigmoid(o_g) * jnp.tanh(cc)
    z = _log_softmax(hh)
    y = jnp.dot(z + xs_ref[...], fw_ref[...],
                preferred_element_type=jnp.float32) + fb_ref[...]
    out_ref[...] = _log_softmax(y)


def _lstm_call(gs, whh_t, bhh, h0, c0, xs, f_w, f_b):
    grid = (NP // BM,)
    return pl.pallas_call(
        _lstm_body,
        grid=grid,
        in_specs=[pl.BlockSpec((BM, 4 * HG), lambda i: (i, 0))] * T + [
            pl.BlockSpec((HG, 4 * HG), lambda i: (0, 0)),
            pl.BlockSpec((1, 4 * HG), lambda i: (0, 0)),
            pl.BlockSpec((BM, HG), lambda i: (i, 0)),
            pl.BlockSpec((BM, HG), lambda i: (i, 0)),
            pl.BlockSpec((BM, HG), lambda i: (i, 0)),
            pl.BlockSpec((HG, NCLS), lambda i: (0, 0)),
            pl.BlockSpec((1, NCLS), lambda i: (0, 0)),
        ],
        out_specs=pl.BlockSpec((BM, NCLS), lambda i: (i, 0)),
        out_shape=jax.ShapeDtypeStruct((NP, NCLS), jnp.float32),
    )(*gs, whh_t, bhh, h0, c0, xs, f_w, f_b)


def _pad_rows(x):
    return jnp.pad(x, ((0, NP - N), (0, 0)))


@jax.jit
def kernel(tx0, tx1, tx2, tx3, tei0, tei1, tei2, tei3, s_x, s_edge_index,
           gw00, gw01, gw10, gw11, gw20, gw21, gw30, gw31, Wih, Whh, bih, bhh,
           s_w1, s_b1, s_w2, s_b2, f_w, f_b, h0, c0):
    txs = [_pad_rows(tx0), _pad_rows(tx1), _pad_rows(tx2), _pad_rows(tx3)]
    teis = [tei0.astype(jnp.int32), tei1.astype(jnp.int32),
            tei2.astype(jnp.int32), tei3.astype(jnp.int32)]
    gws = [gw00, gw01, gw10, gw11, gw20, gw21, gw30, gw31]

    xs = _static_call(s_x, s_w1, s_b1.reshape(1, HN), s_w2,
                      s_b2.reshape(1, HG))

    deg_partials = _degree_call(teis[0][1], teis[1][1], teis[2][1],
                                teis[3][1])
    dinvs = _dinv_call(deg_partials)

    wih_t = Wih.T
    bih2 = bih.reshape(1, 4 * HG)
    whh_t = Whh.T
    bhh2 = bhh.reshape(1, 4 * HG)

    gs = []
    for t in range(T):
        src = teis[t][0]
        dst = teis[t][1]
        dinv = dinvs[t]
        hp1 = _mm1_call(txs[t], gws[2 * t], dinv)
        p1 = _prop_call(hp1, src, dst)
        hp2 = _layer2_call(p1, hp1, dinv, gws[2 * t + 1])
        p2 = _prop_call(hp2, src, dst)
        gs.append(_gates_call(p2, hp2, dinv, wih_t, bih2))

    xs_p = _pad_rows(xs)

    out = _lstm_call(gs, whh_t, bhh2, _pad_rows(h0), _pad_rows(c0), xs_p,
                     f_w, f_b.reshape(1, NCLS))
    return out[:N]


# speedup vs baseline: 1.1579x; 1.1579x over previous
import functools

import jax
import jax.numpy as jnp
from jax import lax
from jax.experimental import pallas as pl
from jax.experimental.pallas import tpu as pltpu
from jax.experimental.pallas import tpu_sc as plsc

N = 10000
NP = 10240
DF = 128
HG = 128
HN = 512
NCLS = 16
T = 4
E = 320000

NCORE = 2
NSUB = 16
NW = NCORE * NSUB
EPW = E // NW
CH = 2000
NCHUNK = EPW // CH
CHP = 160
NCHUNKP = E // CHP
ROWS_PER_SUB = NP // NSUB


def _zero_vmem_1d(ref, nwords):
    zero = jnp.zeros((16,), jnp.float32)

    def body(i, _):
        ref[pl.ds(i * 16, 16)] = zero
        return 0

    lax.fori_loop(0, nwords // 16, body, 0)


def _sc_degree(dst0, dst1, dst2, dst3, out_hbm, ones_v, idx_a, idx_b, zbuf,
               a0, a1, a2, a3, sem_a, sem_b):
    cid = lax.axis_index("c")
    sid = lax.axis_index("s")
    wid = sid * NCORE + cid
    accs = [a0, a1, a2, a3]
    dsts = [dst0, dst1, dst2, dst3]
    idxs = (idx_a, idx_b)
    sems = (sem_a, sem_b)

    one = jnp.full((16,), 1.0, jnp.float32)

    def fill_ones(i, _):
        ones_v[pl.ds(i * 16, 16)] = one
        return 0

    lax.fori_loop(0, CH // 16, fill_ones, 0)
    _zero_vmem_1d(zbuf, ROWS_PER_SUB)

    for g in range(T):
        pltpu.sync_copy(zbuf, accs[g].at[pl.ds(sid * ROWS_PER_SUB, ROWS_PER_SUB)])
    plsc.subcore_barrier()

    for g in range(T):
        dst_hbm = dsts[g]
        acc_g = accs[g]

        def load(j, b, dst_hbm=dst_hbm):
            @pl.when(j < NCHUNK)
            def _():
                off = wid * EPW + j * CH
                pltpu.sync_copy(dst_hbm.at[pl.ds(off, CH)], idxs[b])

        def sstart(b, acc_g=acc_g):
            pltpu.async_copy(ones_v, acc_g.at[idxs[b]], sems[b], add=True)

        def swait(b, acc_g=acc_g):
            pltpu.make_async_copy(ones_v, acc_g.at[idxs[b]], sems[b]).wait()

        load(0, 0)

        def outer(j2, _):
            for b in range(2):
                j = j2 * 2 + b

                @pl.when(j < NCHUNK)
                def _():
                    sstart(b)

                @pl.when(jnp.logical_and(j >= 1, j <= NCHUNK))
                def _():
                    swait(1 - b)

                load(j + 1, 1 - b)
            return 0

        lax.fori_loop(0, NCHUNK // 2 + 1, outer, 0)

    plsc.subcore_barrier()
    for g in range(T):
        sl = pl.ds(sid * ROWS_PER_SUB, ROWS_PER_SUB)
        pltpu.sync_copy(accs[g].at[sl], out_hbm.at[cid, g, sl])


_degree_call = functools.partial(
    pl.kernel,
    out_type=jax.ShapeDtypeStruct((NCORE, T, NP), jnp.float32),
    mesh=plsc.VectorSubcoreMesh(core_axis_name="c", subcore_axis_name="s"),
    scratch_types=[
        pltpu.VMEM((CH,), jnp.float32),
        pltpu.VMEM((CH,), jnp.int32),
        pltpu.VMEM((CH,), jnp.int32),
        pltpu.VMEM((ROWS_PER_SUB,), jnp.float32),
    ] + [pltpu.VMEM_SHARED((NP,), jnp.float32) for _ in range(T)]
      + [pltpu.SemaphoreType.DMA, pltpu.SemaphoreType.DMA],
)(_sc_degree)


HH = HG


def _sc_prop(hp_hbm, src_hbm, dst_hbm, out_hbm, src0, src1, dst0, dst1,
             rows0, rows1, zrow, acc, sem0, sem1):
    cid = lax.axis_index("c")
    sid = lax.axis_index("s")
    wid = sid * NCORE + cid

    zero = jnp.zeros((16,), jnp.float32)

    def zbody(i, _):
        zrow[i // (HG // 16), pl.ds((i % (HG // 16)) * 16, 16)] = zero
        return 0

    lax.fori_loop(0, 16 * (HG // 16), zbody, 0)
    base = sid * ROWS_PER_SUB
    for i in range(ROWS_PER_SUB // 16):
        pltpu.sync_copy(zrow, acc.at[pl.ds(base + i * 16, 16), :])
    plsc.subcore_barrier()

    nw = NCHUNKP // NW + jnp.where(wid < NCHUNKP % NW, 1, 0)
    srcs = (src0, src1)
    dsts = (dst0, dst1)
    rows = (rows0, rows1)
    sems = (sem0, sem1)

    def start(i, b):
        @pl.when(i < nw)
        def _():
            off = (wid + i * NW) * CHP
            pltpu.sync_copy(src_hbm.at[pl.ds(off, CHP)], srcs[b])
            pltpu.sync_copy(dst_hbm.at[pl.ds(off, CHP)], dsts[b])
            pltpu.async_copy(hp_hbm.at[srcs[b]], rows[b], sems[b])

    def finish(i, b):
        @pl.when(i < nw)
        def _():
            pltpu.make_async_copy(hp_hbm.at[srcs[b]], rows[b], sems[b]).wait()
            pltpu.sync_copy(rows[b], acc.at[dsts[b]], add=True)

    start(0, 0)
    start(1, 1)

    def outer(i0, _):
        for b in range(2):
            i = i0 * 2 + b
            finish(i, b)
            start(i + 2, b)
        return 0

    nouter = (NCHUNKP // NW + 1 + 1) // 2
    lax.fori_loop(0, nouter, outer, 0)

    plsc.subcore_barrier()
    sl = pl.ds(base, ROWS_PER_SUB)
    pltpu.sync_copy(acc.at[sl, :], out_hbm.at[cid, sl, :])


_prop_call = functools.partial(
    pl.kernel,
    out_type=jax.ShapeDtypeStruct((NCORE, NP, HG), jnp.float32),
    mesh=plsc.VectorSubcoreMesh(core_axis_name="c", subcore_axis_name="s"),
    scratch_types=[
        pltpu.VMEM((CHP,), jnp.int32),
        pltpu.VMEM((CHP,), jnp.int32),
        pltpu.VMEM((CHP,), jnp.int32),
        pltpu.VMEM((CHP,), jnp.int32),
        pltpu.VMEM((CHP, HG), jnp.float32),
        pltpu.VMEM((CHP, HG), jnp.float32),
        pltpu.VMEM((16, HG), jnp.float32),
        pltpu.VMEM_SHARED((NP, HG), jnp.float32),
        pltpu.SemaphoreType.DMA,
        pltpu.SemaphoreType.DMA,
    ],
)(_sc_prop)


BM = 1024


def _dinv_body(degp_ref, d0, d1, d2, d3):
    deg = degp_ref[0] + degp_ref[1] + 1.0
    dinv = lax.rsqrt(deg)
    outs = [d0, d1, d2, d3]
    for g in range(T):
        outs[g][...] = dinv[g].reshape(NP, 1)


def _dinv_call(deg_partials):
    return pl.pallas_call(
        _dinv_body,
        out_shape=[jax.ShapeDtypeStruct((NP, 1), jnp.float32)] * T,
    )(deg_partials)


def _mm1_body(x_ref, w_ref, dinv_ref, out_ref):
    y = jnp.dot(x_ref[...], w_ref[...], preferred_element_type=jnp.float32)
    out_ref[...] = y * dinv_ref[...]


def _mm1_call(x, w, dinv):
    grid = (NP // BM,)
    return pl.pallas_call(
        _mm1_body,
        grid=grid,
        in_specs=[
            pl.BlockSpec((BM, DF), lambda i: (i, 0)),
            pl.BlockSpec((DF, HG), lambda i: (0, 0)),
            pl.BlockSpec((BM, 1), lambda i: (i, 0)),
        ],
        out_specs=pl.BlockSpec((BM, HG), lambda i: (i, 0)),
        out_shape=jax.ShapeDtypeStruct((NP, HG), jnp.float32),
    )(x, w, dinv)


def _layer2_body(part_ref, hp_ref, dinv_ref, w_ref, out_ref):
    dinv = dinv_ref[...]
    x = (part_ref[0] + part_ref[1] + hp_ref[...]) * dinv
    x = jnp.maximum(x, 0.0)
    y = jnp.dot(x, w_ref[...], preferred_element_type=jnp.float32)
    out_ref[...] = y * dinv


def _layer2_call(partials, hp, dinv, w):
    grid = (NP // BM,)
    return pl.pallas_call(
        _layer2_body,
        grid=grid,
        in_specs=[
            pl.BlockSpec((NCORE, BM, HH), lambda i: (0, i, 0)),
            pl.BlockSpec((BM, HG), lambda i: (i, 0)),
            pl.BlockSpec((BM, 1), lambda i: (i, 0)),
            pl.BlockSpec((HG, HG), lambda i: (0, 0)),
        ],
        out_specs=pl.BlockSpec((BM, HG), lambda i: (i, 0)),
        out_shape=jax.ShapeDtypeStruct((NP, HG), jnp.float32),
    )(partials, hp, dinv, w)


def _gates_body(part_ref, hp_ref, dinv_ref, wih_ref, bih_ref, out_ref):
    x = (part_ref[0] + part_ref[1] + hp_ref[...]) * dinv_ref[...]
    out_ref[...] = (
        jnp.dot(x, wih_ref[...], preferred_element_type=jnp.float32)
        + bih_ref[...]
    )


def _gates_call(partials, hp, dinv, wih_t, bih):
    grid = (NP // BM,)
    return pl.pallas_call(
        _gates_body,
        grid=grid,
        in_specs=[
            pl.BlockSpec((NCORE, BM, HH), lambda i: (0, i, 0)),
            pl.BlockSpec((BM, HG), lambda i: (i, 0)),
            pl.BlockSpec((BM, 1), lambda i: (i, 0)),
            pl.BlockSpec((HG, 4 * HG), lambda i: (0, 0)),
            pl.BlockSpec((1, 4 * HG), lambda i: (0, 0)),
        ],
        out_specs=pl.BlockSpec((BM, 4 * HG), lambda i: (i, 0)),
        out_shape=jax.ShapeDtypeStruct((NP, 4 * HG), jnp.float32),
    )(partials, hp, dinv, wih_t, bih)


def _log_softmax(x):
    m = jnp.max(x, axis=-1, keepdims=True)
    s = x - m
    return s - jnp.log(jnp.sum(jnp.exp(s), axis=-1, keepdims=True))


SBM = 400


def _static_body(sx_ref, w1_ref, b1_ref, w2_ref, b2_ref, out_ref):
    acc = jnp.dot(sx_ref[...], w1_ref[...], preferred_element_type=jnp.float32)
    h = jnp.maximum(acc + b1_ref[...], 0.0)
    y = jnp.dot(h, w2_ref[...], preferred_element_type=jnp.float32)
    y = y + b2_ref[...]
    out_ref[...] = _log_softmax(y)


def _static_call(s_x, w1, b1, w2, b2):
    grid = (N // SBM,)
    return pl.pallas_call(
        _static_body,
        grid=grid,
        in_specs=[
            pl.BlockSpec((SBM, N), lambda i: (i, 0)),
            pl.BlockSpec((N, HN), lambda i: (0, 0)),
            pl.BlockSpec((1, HN), lambda i: (0, 0)),
            pl.BlockSpec((HN, HG), lambda i: (0, 0)),
            pl.BlockSpec((1, HG), lambda i: (0, 0)),
        ],
        out_specs=pl.BlockSpec((SBM, HG), lambda i: (i, 0)),
        out_shape=jax.ShapeDtypeStruct((N, HG), jnp.float32),
    )(s_x, w1, b1, w2, b2)


def _sigmoid(x):
    return 1.0 / (1.0 + jnp.exp(-x))


def _lstm_body(g0, g1, g2, g3, whh_ref, bhh_ref, h0_ref, c0_ref, xs_ref,
               fw_ref, fb_ref, out_ref):
    hh = h0_ref[...]
    cc = c0_ref[...]
    whh = whh_ref[...]
    bhh = bhh_ref[...]
    for g_ref in (g0, g1, g2, g3):
        gates = g_ref[...] + jnp.dot(hh, whh,
                                     preferred_element_type=jnp.float32) + bhh
        i_g = gates[:, 0:HG]
        f_g = gates[:, HG:2 * HG]
        g_g = gates[:, 2 * HG:3 * HG]
        o_g = gates[:, 3 * HG:4 * HG]
        cc = _sigmoid(f_g) * cc + _sigmoid(i_g) * jnp.tanh(g_g)
        hh = _sigmoid(o_g) * jnp.tanh(cc)
    z = _log_softmax(hh)
    y = jnp.dot(z + xs_ref[...], fw_ref[...],
                preferred_element_type=jnp.float32) + fb_ref[...]
    out_ref[...] = _log_softmax(y)


def _lstm_call(gs, whh_t, bhh, h0, c0, xs, f_w, f_b):
    grid = (NP // BM,)
    return pl.pallas_call(
        _lstm_body,
        grid=grid,
        in_specs=[pl.BlockSpec((BM, 4 * HG), lambda i: (i, 0))] * T + [
            pl.BlockSpec((HG, 4 * HG), lambda i: (0, 0)),
            pl.BlockSpec((1, 4 * HG), lambda i: (0, 0)),
            pl.BlockSpec((BM, HG), lambda i: (i, 0)),
            pl.BlockSpec((BM, HG), lambda i: (i, 0)),
            pl.BlockSpec((BM, HG), lambda i: (i, 0)),
            pl.BlockSpec((HG, NCLS), lambda i: (0, 0)),
            pl.BlockSpec((1, NCLS), lambda i: (0, 0)),
        ],
        out_specs=pl.BlockSpec((BM, NCLS), lambda i: (i, 0)),
        out_shape=jax.ShapeDtypeStruct((NP, NCLS), jnp.float32),
    )(*gs, whh_t, bhh, h0, c0, xs, f_w, f_b)


def _pad_rows(x):
    return jnp.pad(x, ((0, NP - N), (0, 0)))


@jax.jit
def kernel(tx0, tx1, tx2, tx3, tei0, tei1, tei2, tei3, s_x, s_edge_index,
           gw00, gw01, gw10, gw11, gw20, gw21, gw30, gw31, Wih, Whh, bih, bhh,
           s_w1, s_b1, s_w2, s_b2, f_w, f_b, h0, c0):
    txs = [_pad_rows(tx0), _pad_rows(tx1), _pad_rows(tx2), _pad_rows(tx3)]
    teis = [tei0.astype(jnp.int32), tei1.astype(jnp.int32),
            tei2.astype(jnp.int32), tei3.astype(jnp.int32)]
    gws = [gw00, gw01, gw10, gw11, gw20, gw21, gw30, gw31]

    deg_partials = _degree_call(teis[0][1], teis[1][1], teis[2][1],
                                teis[3][1])
    dinvs = _dinv_call(deg_partials)

    wih_t = Wih.T
    bih2 = bih.reshape(1, 4 * HG)
    whh_t = Whh.T
    bhh2 = bhh.reshape(1, 4 * HG)

    gs = []
    for t in range(T):
        src = teis[t][0]
        dst = teis[t][1]
        dinv = dinvs[t]
        hp1 = _mm1_call(txs[t], gws[2 * t], dinv)
        p1 = _prop_call(hp1, src, dst)
        hp2 = _layer2_call(p1, hp1, dinv, gws[2 * t + 1])
        p2 = _prop_call(hp2, src, dst)
        gs.append(_gates_call(p2, hp2, dinv, wih_t, bih2))

    xs = _static_call(s_x, s_w1, s_b1.reshape(1, HN), s_w2,
                      s_b2.reshape(1, HG))
    xs_p = _pad_rows(xs)

    out = _lstm_call(gs, whh_t, bhh2, _pad_rows(h0), _pad_rows(c0), xs_p,
                     f_w, f_b.reshape(1, NCLS))
    return out[:N]
